# initial kernel scaffold (unmeasured)
import functools

import jax
import jax.numpy as jnp
from jax import lax
from jax.experimental import pallas as pl
from jax.experimental.pallas import tpu as pltpu

N_DEV = 4


def kernel(x, router_W, route_idx, expert_W):
    n_tok, d = x.shape
    e_local = expert_W.shape[0]
    e_total = router_W.shape[1]
    chunk = n_tok // N_DEV

    def body(x_ref, rw_ref, idx_ref, ew_ref, out_ref,
             gates_ref, send_ref, recv_ref, send_sem, recv_sems):
        my = lax.axis_index("i")
        right = lax.rem(my + 1, N_DEV)
        left = lax.rem(my + N_DEV - 1, N_DEV)

        scores = jnp.dot(x_ref[...].astype(jnp.bfloat16),
                         rw_ref[...].astype(jnp.bfloat16),
                         preferred_element_type=jnp.float32)
        e0 = idx_ref[:, 0:1]
        e1 = idx_ref[:, 1:2]
        eids = lax.broadcasted_iota(jnp.int32, (n_tok, e_total), 1)
        s0 = jnp.sum(jnp.where(eids == e0, scores, 0.0), axis=1, keepdims=True)
        s1 = jnp.sum(jnp.where(eids == e1, scores, 0.0), axis=1, keepdims=True)
        m = jnp.maximum(s0, s1)
        p0 = jnp.exp(s0 - m)
        p1 = jnp.exp(s1 - m)
        g0 = p0 / (p0 + p1)
        g1 = p1 / (p0 + p1)
        jglob = my * e_local + lax.broadcasted_iota(
            jnp.int32, (n_tok, e_local), 1)
        gates_ref[...] = (jnp.where(jglob == e0, g0, 0.0)
                          + jnp.where(jglob == e1, g1, 0.0))

        def partial_chunk(c):
            xs = x_ref[pl.ds(c * chunk, chunk), :].astype(jnp.bfloat16)
            g = gates_ref[pl.ds(c * chunk, chunk), :]
            acc = jnp.zeros((chunk, d), jnp.float32)
            for j in range(e_local):
                y = jnp.dot(xs, ew_ref[j].astype(jnp.bfloat16),
                            preferred_element_type=jnp.float32)
                acc = acc + g[:, j:j + 1] * y
            return acc

        send_ref[...] = partial_chunk(lax.rem(my + N_DEV - 1, N_DEV))

        barrier_sem = pltpu.get_barrier_semaphore()
        for nbr in (left, right):
            pl.semaphore_signal(barrier_sem, inc=1, device_id=(nbr,),
                                device_id_type=pl.DeviceIdType.MESH)
        pl.semaphore_wait(barrier_sem, 2)

        for h in range(N_DEV - 1):
            rdma = pltpu.make_async_remote_copy(
                src_ref=send_ref,
                dst_ref=recv_ref.at[h],
                send_sem=send_sem,
                recv_sem=recv_sems.at[h],
                device_id=(right,),
                device_id_type=pl.DeviceIdType.MESH,
            )
            rdma.start()
            acc = partial_chunk(lax.rem(my + 2 * N_DEV - 2 - h, N_DEV))
            rdma.wait()
            if h < N_DEV - 2:
                send_ref[...] = acc + recv_ref[h]
            else:
                out_ref[...] = acc + recv_ref[h]

        @functools.partial(pl.run_scoped,
                           exit_sem=pltpu.SemaphoreType.REGULAR)
        def _(exit_sem):
            for nbr in (left, right):
                pl.semaphore_signal(exit_sem, inc=1, device_id=(nbr,),
                                    device_id_type=pl.DeviceIdType.MESH)
            pl.semaphore_wait(exit_sem, 2)

    return pl.pallas_call(
        body,
        out_shape=jax.ShapeDtypeStruct((chunk, d), jnp.float32),
        in_specs=[pl.BlockSpec(memory_space=pltpu.VMEM)] * 4,
        out_specs=pl.BlockSpec(memory_space=pltpu.VMEM),
        scratch_shapes=[
            pltpu.VMEM((n_tok, e_local), jnp.float32),
            pltpu.VMEM((chunk, d), jnp.float32),
            pltpu.VMEM((N_DEV - 1, chunk, d), jnp.float32),
            pltpu.SemaphoreType.DMA,
            pltpu.SemaphoreType.DMA((N_DEV - 1,)),
        ],
        compiler_params=pltpu.CompilerParams(collective_id=0),
    )(x, router_W, route_idx, expert_W)


# baseline (device time: 109939 ns/iter reference)
import functools

import jax
import jax.numpy as jnp
from jax import lax
from jax.experimental import pallas as pl
from jax.experimental.pallas import tpu as pltpu

N_DEV = 4


def kernel(x, router_W, route_idx, expert_W):
    n_tok, d = x.shape
    e_local = expert_W.shape[0]
    e_total = router_W.shape[1]
    chunk = n_tok // N_DEV

    def body(x_ref, rw_ref, idx_ref, ew_ref, out_ref,
             gates_ref, send_ref, recv_ref, send_sem, recv_sems):
        my = lax.axis_index("i")
        right = lax.rem(my + 1, N_DEV)
        left = lax.rem(my + N_DEV - 1, N_DEV)

        scores = jnp.dot(x_ref[...].astype(jnp.bfloat16),
                         rw_ref[...].astype(jnp.bfloat16),
                         preferred_element_type=jnp.float32)
        e0 = idx_ref[:, 0:1]
        e1 = idx_ref[:, 1:2]
        eids = lax.broadcasted_iota(jnp.int32, (n_tok, e_total), 1)
        s0 = jnp.sum(jnp.where(eids == e0, scores, 0.0), axis=1, keepdims=True)
        s1 = jnp.sum(jnp.where(eids == e1, scores, 0.0), axis=1, keepdims=True)
        m = jnp.maximum(s0, s1)
        p0 = jnp.exp(s0 - m)
        p1 = jnp.exp(s1 - m)
        g0 = p0 / (p0 + p1)
        g1 = p1 / (p0 + p1)
        jglob = my * e_local + lax.broadcasted_iota(
            jnp.int32, (n_tok, e_local), 1)
        gates_ref[...] = (jnp.where(jglob == e0, g0, 0.0)
                          + jnp.where(jglob == e1, g1, 0.0))

        def partial_chunk(c):
            xs = x_ref[pl.ds(c * chunk, chunk), :].astype(jnp.bfloat16)
            g = gates_ref[pl.ds(c * chunk, chunk), :]
            acc = jnp.zeros((chunk, d), jnp.float32)
            for j in range(e_local):
                y = jnp.dot(xs, ew_ref[j].astype(jnp.bfloat16),
                            preferred_element_type=jnp.float32)
                acc = acc + g[:, j:j + 1] * y
            return acc

        send_ref[...] = partial_chunk(lax.rem(my + N_DEV - 1, N_DEV))

        barrier_sem = pltpu.get_barrier_semaphore()
        for nbr in (left, right):
            pl.semaphore_signal(barrier_sem, inc=1, device_id=(nbr,),
                                device_id_type=pl.DeviceIdType.MESH)
        pl.semaphore_wait(barrier_sem, 2)

        for h in range(N_DEV - 1):
            rdma = pltpu.make_async_remote_copy(
                src_ref=send_ref,
                dst_ref=recv_ref.at[h],
                send_sem=send_sem,
                recv_sem=recv_sems.at[h],
                device_id=(right,),
                device_id_type=pl.DeviceIdType.MESH,
            )
            rdma.start()
            acc = partial_chunk(lax.rem(my + 2 * N_DEV - 2 - h, N_DEV))
            rdma.wait()
            if h < N_DEV - 2:
                send_ref[...] = acc + recv_ref[h]
            else:
                out_ref[...] = acc + recv_ref[h]

        @functools.partial(pl.run_scoped,
                           exit_sem=pltpu.SemaphoreType.REGULAR)
        def _(exit_sem):
            for nbr in (left, right):
                pl.semaphore_signal(exit_sem, inc=1, device_id=(nbr,),
                                    device_id_type=pl.DeviceIdType.MESH)
            pl.semaphore_wait(exit_sem, 2)

    return pl.pallas_call(
        body,
        out_shape=jax.ShapeDtypeStruct((chunk, d), jnp.float32),
        in_specs=[pl.BlockSpec(memory_space=pltpu.VMEM)] * 4,
        out_specs=pl.BlockSpec(memory_space=pltpu.VMEM),
        scratch_shapes=[
            pltpu.VMEM((n_tok, e_local), jnp.float32),
            pltpu.VMEM((chunk, d), jnp.float32),
            pltpu.VMEM((N_DEV - 1, chunk, d), jnp.float32),
            pltpu.SemaphoreType.DMA,
            pltpu.SemaphoreType.DMA((N_DEV - 1,)),
        ],
        compiler_params=pltpu.CompilerParams(
            collective_id=0, vmem_limit_bytes=100 * 1024 * 1024),
    )(x, router_W, route_idx, expert_W)


# device time: 76157 ns/iter; 1.4436x vs baseline; 1.4436x over previous
import functools

import jax
import jax.numpy as jnp
from jax import lax
from jax.experimental import pallas as pl
from jax.experimental.pallas import tpu as pltpu

N_DEV = 4


def kernel(x, router_W, route_idx, expert_W):
    n_tok, d = x.shape
    e_local = expert_W.shape[0]
    e_total = router_W.shape[1]
    chunk = n_tok // N_DEV

    def body(x_ref, rw_ref, idx_ref, ew_ref, out_ref,
             gates_ref, send_ref, recv_ref, send_sem, recv_sems):
        my = lax.axis_index("i")
        right = lax.rem(my + 1, N_DEV)
        left = lax.rem(my + N_DEV - 1, N_DEV)

        scores = jnp.dot(x_ref[...].astype(jnp.bfloat16),
                         rw_ref[...].astype(jnp.bfloat16),
                         preferred_element_type=jnp.float32)
        e0 = idx_ref[:, 0:1]
        e1 = idx_ref[:, 1:2]
        eids = lax.broadcasted_iota(jnp.int32, (n_tok, e_total), 1)
        s0 = jnp.sum(jnp.where(eids == e0, scores, 0.0), axis=1, keepdims=True)
        s1 = jnp.sum(jnp.where(eids == e1, scores, 0.0), axis=1, keepdims=True)
        m = jnp.maximum(s0, s1)
        p0 = jnp.exp(s0 - m)
        p1 = jnp.exp(s1 - m)
        g0 = p0 / (p0 + p1)
        g1 = p1 / (p0 + p1)
        jglob = my * e_local + lax.broadcasted_iota(
            jnp.int32, (n_tok, e_local), 1)
        gates_ref[...] = (jnp.where(jglob == e0, g0, 0.0)
                          + jnp.where(jglob == e1, g1, 0.0))

        def partial_chunk(c):
            xs = x_ref[pl.ds(c * chunk, chunk), :].astype(jnp.bfloat16)
            g = gates_ref[pl.ds(c * chunk, chunk), :]
            acc = jnp.zeros((chunk, d), jnp.float32)
            for j in range(e_local):
                y = jnp.dot(xs, ew_ref[j].astype(jnp.bfloat16),
                            preferred_element_type=jnp.float32)
                acc = acc + g[:, j:j + 1] * y
            return acc

        send_ref[...] = partial_chunk(
            lax.rem(my + N_DEV - 1, N_DEV)).astype(jnp.bfloat16)

        barrier_sem = pltpu.get_barrier_semaphore()
        for nbr in (left, right):
            pl.semaphore_signal(barrier_sem, inc=1, device_id=(nbr,),
                                device_id_type=pl.DeviceIdType.MESH)
        pl.semaphore_wait(barrier_sem, 2)

        for h in range(N_DEV - 1):
            rdma = pltpu.make_async_remote_copy(
                src_ref=send_ref,
                dst_ref=recv_ref.at[h],
                send_sem=send_sem,
                recv_sem=recv_sems.at[h],
                device_id=(right,),
                device_id_type=pl.DeviceIdType.MESH,
            )
            rdma.start()
            acc = partial_chunk(lax.rem(my + 2 * N_DEV - 2 - h, N_DEV))
            rdma.wait()
            if h < N_DEV - 2:
                send_ref[...] = (
                    acc + recv_ref[h].astype(jnp.float32)
                ).astype(jnp.bfloat16)
            else:
                out_ref[...] = acc + recv_ref[h].astype(jnp.float32)

        @functools.partial(pl.run_scoped,
                           exit_sem=pltpu.SemaphoreType.REGULAR)
        def _(exit_sem):
            for nbr in (left, right):
                pl.semaphore_signal(exit_sem, inc=1, device_id=(nbr,),
                                    device_id_type=pl.DeviceIdType.MESH)
            pl.semaphore_wait(exit_sem, 2)

    return pl.pallas_call(
        body,
        out_shape=jax.ShapeDtypeStruct((chunk, d), jnp.float32),
        in_specs=[pl.BlockSpec(memory_space=pltpu.VMEM)] * 4,
        out_specs=pl.BlockSpec(memory_space=pltpu.VMEM),
        scratch_shapes=[
            pltpu.VMEM((n_tok, e_local), jnp.float32),
            pltpu.VMEM((chunk, d), jnp.bfloat16),
            pltpu.VMEM((N_DEV - 1, chunk, d), jnp.bfloat16),
            pltpu.SemaphoreType.DMA,
            pltpu.SemaphoreType.DMA((N_DEV - 1,)),
        ],
        compiler_params=pltpu.CompilerParams(
            collective_id=0, vmem_limit_bytes=100 * 1024 * 1024),
    )(x, router_W, route_idx, expert_W)


# device time: 74110 ns/iter; 1.4835x vs baseline; 1.0276x over previous
import jax
import jax.numpy as jnp
from jax import lax
from jax.experimental import pallas as pl
from jax.experimental.pallas import tpu as pltpu

N_DEV = 4


def kernel(x, router_W, route_idx, expert_W):
    n_tok, d = x.shape
    e_local = expert_W.shape[0]
    e_total = router_W.shape[1]
    chunk = n_tok // N_DEV

    def body(x_ref, rw_ref, idx_ref, ew_ref, out_ref,
             gates_ref, send_ref, recv_ref, send_sems, recv_sems):
        my = lax.axis_index("i")

        scores = jnp.dot(x_ref[...].astype(jnp.bfloat16),
                         rw_ref[...].astype(jnp.bfloat16),
                         preferred_element_type=jnp.float32)
        e0 = idx_ref[:, 0:1]
        e1 = idx_ref[:, 1:2]
        eids = lax.broadcasted_iota(jnp.int32, (n_tok, e_total), 1)
        s0 = jnp.sum(jnp.where(eids == e0, scores, 0.0), axis=1, keepdims=True)
        s1 = jnp.sum(jnp.where(eids == e1, scores, 0.0), axis=1, keepdims=True)
        m = jnp.maximum(s0, s1)
        p0 = jnp.exp(s0 - m)
        p1 = jnp.exp(s1 - m)
        g0 = p0 / (p0 + p1)
        g1 = p1 / (p0 + p1)
        jglob = my * e_local + lax.broadcasted_iota(
            jnp.int32, (n_tok, e_local), 1)
        gates_ref[...] = (jnp.where(jglob == e0, g0, 0.0)
                          + jnp.where(jglob == e1, g1, 0.0))

        def partial_chunk(c):
            xs = x_ref[pl.ds(c * chunk, chunk), :].astype(jnp.bfloat16)
            g = gates_ref[pl.ds(c * chunk, chunk), :]
            acc = jnp.zeros((chunk, d), jnp.float32)
            for j in range(e_local):
                y = jnp.dot(xs, ew_ref[j].astype(jnp.bfloat16),
                            preferred_element_type=jnp.float32)
                acc = acc + g[:, j:j + 1] * y
            return acc

        barrier_sem = pltpu.get_barrier_semaphore()
        for off in range(1, N_DEV):
            pl.semaphore_signal(
                barrier_sem, inc=1,
                device_id=(lax.rem(my + off, N_DEV),),
                device_id_type=pl.DeviceIdType.MESH)
        pl.semaphore_wait(barrier_sem, N_DEV - 1)

        rdmas = []
        for k in range(N_DEV - 1):
            c = lax.rem(my + 1 + k, N_DEV)
            send_ref[k] = partial_chunk(c).astype(jnp.bfloat16)
            rdma = pltpu.make_async_remote_copy(
                src_ref=send_ref.at[k],
                dst_ref=recv_ref.at[2 - k],
                send_sem=send_sems.at[k],
                recv_sem=recv_sems.at[2 - k],
                device_id=(c,),
                device_id_type=pl.DeviceIdType.MESH,
            )
            rdma.start()
            rdmas.append(rdma)

        acc = partial_chunk(my)
        for rdma in rdmas:
            rdma.wait_send()
            rdma.wait_recv()
        out_ref[...] = (acc
                        + recv_ref[0].astype(jnp.float32)
                        + recv_ref[1].astype(jnp.float32)
                        + recv_ref[2].astype(jnp.float32))

    return pl.pallas_call(
        body,
        out_shape=jax.ShapeDtypeStruct((chunk, d), jnp.float32),
        in_specs=[pl.BlockSpec(memory_space=pltpu.VMEM)] * 4,
        out_specs=pl.BlockSpec(memory_space=pltpu.VMEM),
        scratch_shapes=[
            pltpu.VMEM((n_tok, e_local), jnp.float32),
            pltpu.VMEM((N_DEV - 1, chunk, d), jnp.bfloat16),
            pltpu.VMEM((N_DEV - 1, chunk, d), jnp.bfloat16),
            pltpu.SemaphoreType.DMA((N_DEV - 1,)),
            pltpu.SemaphoreType.DMA((N_DEV - 1,)),
        ],
        compiler_params=pltpu.CompilerParams(
            collective_id=0, vmem_limit_bytes=100 * 1024 * 1024),
    )(x, router_W, route_idx, expert_W)
